# Initial kernel scaffold; baseline (speedup 1.0000x reference)
import jax
import jax.numpy as jnp
from jax.experimental import pallas as pl


def kernel(ids, feats, layer_idx, node_table, node_fc_w, node_fc_b, feat_table, feat_fc_w, feat_fc_b):
    raise NotImplementedError("write your pallas kernel here")


# speedup vs baseline: 8.5272x; 8.5272x over previous
import functools

import jax
import jax.numpy as jnp
from jax import lax
from jax.experimental import pallas as pl
from jax.experimental.pallas import tpu as pltpu
from jax.experimental.pallas import tpu_sc as plsc

_B = 4096
_L = 200
_D = 32
_NC = 2
_NS = 16
_NW = _NC * _NS
_BPW = _B // _NW
_CH0 = 104
_CH1 = _L - _CH0

_mesh = plsc.VectorSubcoreMesh(core_axis_name="c", subcore_axis_name="s")


def _sc_body(feats_hbm, nidx_hbm, ftab_hbm, ntab_hbm, fsum_hbm, nrow_hbm,
             fidx_v, nidx_v, rows_v, facc_v, nrow_v, sem_f, sem_n):
    wid = lax.axis_index("s") * _NC + lax.axis_index("c")
    base = wid * _BPW
    pltpu.sync_copy(feats_hbm.at[pl.ds(base, _BPW), :], fidx_v)
    pltpu.sync_copy(nidx_hbm.at[pl.ds(base, _BPW)], nidx_v)
    ncp = pltpu.async_copy(ntab_hbm.at[nidx_v], nrow_v, sem_n)

    def bag(b, carry):
        cp1 = pltpu.async_copy(ftab_hbm.at[fidx_v.at[b, pl.ds(0, _CH0)]],
                               rows_v.at[pl.ds(0, _CH0), :], sem_f)
        cp2 = pltpu.async_copy(ftab_hbm.at[fidx_v.at[b, pl.ds(_CH0, _CH1)]],
                               rows_v.at[pl.ds(_CH0, _CH1), :], sem_f)
        cp1.wait()
        cp2.wait()

        def red(j, acc):
            a0, a1 = acc
            return (a0 + rows_v[j, pl.ds(0, 16)],
                    a1 + rows_v[j, pl.ds(16, 16)])

        a0, a1 = lax.fori_loop(
            0, _L, red,
            (jnp.zeros((16,), jnp.float32), jnp.zeros((16,), jnp.float32)))
        facc_v[b, pl.ds(0, 16)] = a0
        facc_v[b, pl.ds(16, 16)] = a1
        return carry

    lax.fori_loop(0, _BPW, bag, 0)
    ncp.wait()
    pltpu.sync_copy(facc_v, fsum_hbm.at[pl.ds(base, _BPW), :])
    pltpu.sync_copy(nrow_v, nrow_hbm.at[pl.ds(base, _BPW), :])


_sc_pool = functools.partial(
    pl.kernel,
    out_type=(jax.ShapeDtypeStruct((_B, _D), jnp.float32),
              jax.ShapeDtypeStruct((_B, _D), jnp.float32)),
    mesh=_mesh,
    scratch_types=[
        pltpu.VMEM((_BPW, _L), jnp.int32),
        pltpu.VMEM((_BPW,), jnp.int32),
        pltpu.VMEM((_L, _D), jnp.float32),
        pltpu.VMEM((_BPW, _D), jnp.float32),
        pltpu.VMEM((_BPW, _D), jnp.float32),
        pltpu.SemaphoreType.DMA,
        pltpu.SemaphoreType.DMA,
    ],
    compiler_params=pltpu.CompilerParams(use_tc_tiling_on_sc=False),
)(_sc_body)


def _tc_body(fsum_ref, nrow_ref, fw_ref, fb_ref, nw_ref, nb_ref, out_ref):
    fs = fsum_ref[...] * (1.0 / _L)
    fo = lax.dot_general(fs, fw_ref[...], (((1,), (1,)), ((), ())),
                         preferred_element_type=jnp.float32)
    no = lax.dot_general(nrow_ref[...], nw_ref[...], (((1,), (1,)), ((), ())),
                         preferred_element_type=jnp.float32)
    out_ref[:, 0:_D] = fo + fb_ref[...]
    out_ref[:, _D:2 * _D] = no + nb_ref[...]


def kernel(ids, feats, layer_idx, node_table, node_fc_w, node_fc_b,
           feat_table, feat_fc_w, feat_fc_b):
    n_nodes = node_table.shape[0] - 1
    idx = jnp.where(layer_idx > 0, ids,
                    jnp.full_like(ids, n_nodes)).astype(jnp.int32)
    feats = feats.astype(jnp.int32)
    fsum, nrow = _sc_pool(feats, idx, feat_table, node_table)
    out = pl.pallas_call(
        _tc_body,
        out_shape=jax.ShapeDtypeStruct((_B, 2 * _D), jnp.float32),
    )(fsum, nrow, feat_fc_w, feat_fc_b.reshape(1, _D),
      node_fc_w, node_fc_b.reshape(1, _D))
    return out


# speedup vs baseline: 13.4450x; 1.5767x over previous
import functools

import jax
import jax.numpy as jnp
from jax import lax
from jax.experimental import pallas as pl
from jax.experimental.pallas import tpu as pltpu
from jax.experimental.pallas import tpu_sc as plsc

_B = 4096
_L = 200
_D = 32
_NC = 2
_NS = 16
_NW = _NC * _NS
_BPW = _B // _NW
_CH0 = 104
_CH1 = _L - _CH0

_mesh = plsc.VectorSubcoreMesh(core_axis_name="c", subcore_axis_name="s")


def _sc_body(feats_hbm, nidx_hbm, ftab_hbm, ntab_hbm, fsum_hbm, nrow_hbm,
             fidx_v, nidx_v, rows_v, facc_v, nrow_v, sem_a, sem_b, sem_n):
    wid = lax.axis_index("s") * _NC + lax.axis_index("c")
    base = wid * _BPW
    pltpu.sync_copy(feats_hbm.at[pl.ds(base, _BPW), :], fidx_v)
    pltpu.sync_copy(nidx_hbm.at[pl.ds(base, _BPW)], nidx_v)
    ncp = pltpu.async_copy(ntab_hbm.at[nidx_v], nrow_v, sem_n)

    def issue(b, slot, sem):
        bb = jnp.minimum(b, _BPW - 1)
        pltpu.async_copy(ftab_hbm.at[fidx_v.at[bb, pl.ds(0, _CH0)]],
                         rows_v.at[slot, pl.ds(0, _CH0), :], sem)
        pltpu.async_copy(ftab_hbm.at[fidx_v.at[bb, pl.ds(_CH0, _CH1)]],
                         rows_v.at[slot, pl.ds(_CH0, _CH1), :], sem)

    def drain(slot, sem):
        pltpu.make_async_copy(ftab_hbm.at[pl.ds(0, _CH0), :],
                              rows_v.at[slot, pl.ds(0, _CH0), :], sem).wait()
        pltpu.make_async_copy(ftab_hbm.at[pl.ds(0, _CH1), :],
                              rows_v.at[slot, pl.ds(_CH0, _CH1), :], sem).wait()

    def reduce_store(b, slot):
        zeros = jnp.zeros((16,), jnp.float32)

        def red(j, acc):
            accs = list(acc)
            r = j * 4
            for k in range(4):
                accs[2 * k] = accs[2 * k] + rows_v[slot, r + k, pl.ds(0, 16)]
                accs[2 * k + 1] = (accs[2 * k + 1]
                                   + rows_v[slot, r + k, pl.ds(16, 16)])
            return tuple(accs)

        acc = lax.fori_loop(0, _L // 4, red, (zeros,) * 8)
        facc_v[b, pl.ds(0, 16)] = (acc[0] + acc[2]) + (acc[4] + acc[6])
        facc_v[b, pl.ds(16, 16)] = (acc[1] + acc[3]) + (acc[5] + acc[7])

    issue(0, 0, sem_a)

    def pair(i, carry):
        b0 = 2 * i
        issue(b0 + 1, 1, sem_b)
        drain(0, sem_a)
        reduce_store(b0, 0)
        issue(b0 + 2, 0, sem_a)
        drain(1, sem_b)
        reduce_store(b0 + 1, 1)
        return carry

    lax.fori_loop(0, _BPW // 2, pair, 0)
    drain(0, sem_a)
    ncp.wait()
    pltpu.sync_copy(facc_v, fsum_hbm.at[pl.ds(base, _BPW), :])
    pltpu.sync_copy(nrow_v, nrow_hbm.at[pl.ds(base, _BPW), :])


_sc_pool = functools.partial(
    pl.kernel,
    out_type=(jax.ShapeDtypeStruct((_B, _D), jnp.float32),
              jax.ShapeDtypeStruct((_B, _D), jnp.float32)),
    mesh=_mesh,
    scratch_types=[
        pltpu.VMEM((_BPW, _L), jnp.int32),
        pltpu.VMEM((_BPW,), jnp.int32),
        pltpu.VMEM((2, _L, _D), jnp.float32),
        pltpu.VMEM((_BPW, _D), jnp.float32),
        pltpu.VMEM((_BPW, _D), jnp.float32),
        pltpu.SemaphoreType.DMA,
        pltpu.SemaphoreType.DMA,
        pltpu.SemaphoreType.DMA,
    ],
    compiler_params=pltpu.CompilerParams(use_tc_tiling_on_sc=False),
)(_sc_body)


def _tc_body(fsum_ref, nrow_ref, fw_ref, fb_ref, nw_ref, nb_ref, out_ref):
    fs = fsum_ref[...] * (1.0 / _L)
    fo = lax.dot_general(fs, fw_ref[...], (((1,), (1,)), ((), ())),
                         preferred_element_type=jnp.float32)
    no = lax.dot_general(nrow_ref[...], nw_ref[...], (((1,), (1,)), ((), ())),
                         preferred_element_type=jnp.float32)
    out_ref[:, 0:_D] = fo + fb_ref[...]
    out_ref[:, _D:2 * _D] = no + nb_ref[...]


def kernel(ids, feats, layer_idx, node_table, node_fc_w, node_fc_b,
           feat_table, feat_fc_w, feat_fc_b):
    n_nodes = node_table.shape[0] - 1
    idx = jnp.where(layer_idx > 0, ids,
                    jnp.full_like(ids, n_nodes)).astype(jnp.int32)
    feats = feats.astype(jnp.int32)
    fsum, nrow = _sc_pool(feats, idx, feat_table, node_table)
    out = pl.pallas_call(
        _tc_body,
        out_shape=jax.ShapeDtypeStruct((_B, 2 * _D), jnp.float32),
    )(fsum, nrow, feat_fc_w, feat_fc_b.reshape(1, _D),
      node_fc_w, node_fc_b.reshape(1, _D))
    return out


# speedup vs baseline: 15.1824x; 1.1292x over previous
import functools

import jax
import jax.numpy as jnp
from jax import lax
from jax.experimental import pallas as pl
from jax.experimental.pallas import tpu as pltpu
from jax.experimental.pallas import tpu_sc as plsc

_B = 4096
_L = 200
_D = 32
_NC = 2
_NS = 16
_NW = _NC * _NS
_BPW = _B // _NW
_CH0 = 104
_CH1 = _L - _CH0

_mesh = plsc.VectorSubcoreMesh(core_axis_name="c", subcore_axis_name="s")


_NBUF = 4


def _sc_body(feats_hbm, nidx_hbm, ftab_hbm, ntab_hbm, fsum_hbm, nrow_hbm,
             fidx_v, nidx_v, rows_v, facc_v, nrow_v, sems, sem_n):
    wid = lax.axis_index("s") * _NC + lax.axis_index("c")
    base = wid * _BPW
    pltpu.sync_copy(feats_hbm.at[pl.ds(base, _BPW), :], fidx_v)
    pltpu.sync_copy(nidx_hbm.at[pl.ds(base, _BPW)], nidx_v)
    ncp = pltpu.async_copy(ntab_hbm.at[nidx_v], nrow_v, sem_n)

    def issue(b, slot):
        bb = jnp.minimum(b, _BPW - 1)
        pltpu.async_copy(ftab_hbm.at[fidx_v.at[bb, pl.ds(0, _CH0)]],
                         rows_v.at[slot, pl.ds(0, _CH0), :], sems.at[slot])
        pltpu.async_copy(ftab_hbm.at[fidx_v.at[bb, pl.ds(_CH0, _CH1)]],
                         rows_v.at[slot, pl.ds(_CH0, _CH1), :], sems.at[slot])

    def drain(slot):
        pltpu.make_async_copy(ftab_hbm.at[pl.ds(0, _CH0), :],
                              rows_v.at[slot, pl.ds(0, _CH0), :],
                              sems.at[slot]).wait()
        pltpu.make_async_copy(ftab_hbm.at[pl.ds(0, _CH1), :],
                              rows_v.at[slot, pl.ds(_CH0, _CH1), :],
                              sems.at[slot]).wait()

    def reduce_store(b, slot):
        zeros = jnp.zeros((16,), jnp.float32)

        @plsc.parallel_loop(0, _L, step=4, unroll=2, carry=(zeros,) * 8)
        def red(j, accs):
            a = list(accs)
            for k in range(4):
                a[2 * k] = a[2 * k] + rows_v[slot, j + k, pl.ds(0, 16)]
                a[2 * k + 1] = (a[2 * k + 1]
                                + rows_v[slot, j + k, pl.ds(16, 16)])
            return tuple(a)

        acc = red
        facc_v[b, pl.ds(0, 16)] = (acc[0] + acc[2]) + (acc[4] + acc[6])
        facc_v[b, pl.ds(16, 16)] = (acc[1] + acc[3]) + (acc[5] + acc[7])

    for s in range(_NBUF - 1):
        issue(s, s)

    def quad(q, carry):
        b0 = _NBUF * q
        issue(b0 + _NBUF - 1, _NBUF - 1)
        for s in range(_NBUF):
            drain(s)
            reduce_store(b0 + s, s)
            if s < _NBUF - 1:
                issue(b0 + _NBUF + s, s)
        return carry

    lax.fori_loop(0, _BPW // _NBUF, quad, 0)
    for s in range(_NBUF - 1):
        drain(s)
    ncp.wait()
    pltpu.sync_copy(facc_v, fsum_hbm.at[pl.ds(base, _BPW), :])
    pltpu.sync_copy(nrow_v, nrow_hbm.at[pl.ds(base, _BPW), :])


_sc_pool = functools.partial(
    pl.kernel,
    out_type=(jax.ShapeDtypeStruct((_B, _D), jnp.float32),
              jax.ShapeDtypeStruct((_B, _D), jnp.float32)),
    mesh=_mesh,
    scratch_types=[
        pltpu.VMEM((_BPW, _L), jnp.int32),
        pltpu.VMEM((_BPW,), jnp.int32),
        pltpu.VMEM((_NBUF, _L, _D), jnp.float32),
        pltpu.VMEM((_BPW, _D), jnp.float32),
        pltpu.VMEM((_BPW, _D), jnp.float32),
        pltpu.SemaphoreType.DMA((_NBUF,)),
        pltpu.SemaphoreType.DMA,
    ],
    compiler_params=pltpu.CompilerParams(use_tc_tiling_on_sc=False),
)(_sc_body)


def _tc_body(fsum_ref, nrow_ref, fw_ref, fb_ref, nw_ref, nb_ref, out_ref):
    fs = fsum_ref[...] * (1.0 / _L)
    fo = lax.dot_general(fs, fw_ref[...], (((1,), (1,)), ((), ())),
                         preferred_element_type=jnp.float32)
    no = lax.dot_general(nrow_ref[...], nw_ref[...], (((1,), (1,)), ((), ())),
                         preferred_element_type=jnp.float32)
    out_ref[:, 0:_D] = fo + fb_ref[...]
    out_ref[:, _D:2 * _D] = no + nb_ref[...]


def kernel(ids, feats, layer_idx, node_table, node_fc_w, node_fc_b,
           feat_table, feat_fc_w, feat_fc_b):
    n_nodes = node_table.shape[0] - 1
    idx = jnp.where(layer_idx > 0, ids,
                    jnp.full_like(ids, n_nodes)).astype(jnp.int32)
    feats = feats.astype(jnp.int32)
    fsum, nrow = _sc_pool(feats, idx, feat_table, node_table)
    out = pl.pallas_call(
        _tc_body,
        out_shape=jax.ShapeDtypeStruct((_B, 2 * _D), jnp.float32),
    )(fsum, nrow, feat_fc_w, feat_fc_b.reshape(1, _D),
      node_fc_w, node_fc_b.reshape(1, _D))
    return out
